# Initial kernel scaffold; baseline (speedup 1.0000x reference)
import jax
import jax.numpy as jnp
from jax.experimental import pallas as pl


def kernel(gt, pred):
    raise NotImplementedError("write your pallas kernel here")


# speedup vs baseline: 100.5950x; 100.5950x over previous
import dataclasses
import functools

import jax
import jax.numpy as jnp
from jax import lax
from jax.experimental import pallas as pl
from jax.experimental.pallas import tpu as pltpu
from jax.experimental.pallas import tpu_sc as plsc

NL = 20
NPIX = 1024 * 1024
NC, NS, L = 2, 16, 16
NW = NC * NS
PER_W = NPIX // NW
NVEC = PER_W // L
NBINS = NL * NL


HC = NL * L


def _hist_sc(gt_flat, pred_flat):
    mesh = plsc.VectorSubcoreMesh(core_axis_name="c", subcore_axis_name="s")
    cp = pltpu.CompilerParams()
    if "needs_layout_passes" in pltpu.CompilerParams.__dataclass_fields__:
        cp = dataclasses.replace(cp, needs_layout_passes=False)

    @functools.partial(
        pl.kernel,
        out_type=jax.ShapeDtypeStruct((NW, NL, HC), jnp.int32),
        mesh=mesh,
        compiler_params=cp,
        scratch_types=[
            pltpu.VMEM((PER_W,), jnp.int32),
            pltpu.VMEM((PER_W,), jnp.int32),
            pltpu.VMEM((NL, HC), jnp.int32),
        ],
    )
    def hist_kernel(gt_hbm, pred_hbm, out_hbm, gt_v, pred_v, hist_v):
        wid = lax.axis_index("s") * NC + lax.axis_index("c")
        base = wid * PER_W

        zeros = jnp.zeros((L,), jnp.int32)

        @pl.loop(0, NL)
        def _(g):
            @pl.loop(0, NL)
            def _(c):
                hist_v[g, pl.ds(c * L, L)] = zeros

        pltpu.sync_copy(gt_hbm.at[pl.ds(base, PER_W)], gt_v)
        pltpu.sync_copy(pred_hbm.at[pl.ds(base, PER_W)], pred_v)

        lane = lax.iota(jnp.int32, L)
        ones = jnp.ones((L,), jnp.int32)

        @pl.loop(0, NVEC)
        def _(i):
            g = gt_v[pl.ds(i * L, L)]
            p = pred_v[pl.ds(i * L, L)]
            col = p * L + lane
            plsc.addupdate_scatter(hist_v, [g, col], ones)

        pltpu.sync_copy(hist_v, out_hbm.at[wid])

    return hist_kernel(gt_flat, pred_flat)


def _sim_body(parts_ref, out_ref):
    x = parts_ref[...]
    H = jnp.sum(x, axis=0).astype(jnp.float32)
    sel = (lax.broadcasted_iota(jnp.int32, (HC, NL), 0) // L
           == lax.broadcasted_iota(jnp.int32, (HC, NL), 1)
           ).astype(jnp.float32)
    M0 = jnp.dot(H, sel,
                 preferred_element_type=jnp.float32).astype(jnp.int32)
    rows0 = jnp.sum(M0, axis=1, keepdims=True)

    ci = lax.broadcasted_iota(jnp.int32, (1, NL), 1)
    ri = lax.broadcasted_iota(jnp.int32, (NL, 1), 0)
    col0_i = jnp.where(ci == 0, 1, 0)
    row0_i = jnp.where(ri == 0, 1, 0)
    n_total = jnp.int32(NPIX)
    one_f = jnp.float32(1.0)
    zero_f = jnp.float32(0.0)

    def outer(label, carry):
        M, tp, iou = carry
        is_row = jnp.where(ri == label, 1, 0)
        row = jnp.sum(M * is_row, axis=0, keepdims=True)
        S = jnp.sum(row)
        col = jnp.sum(M, axis=0, keepdims=True)
        lp_i = jnp.where(jnp.sum(rows0 * is_row) > 0, 1, 0)

        inter_f = row.astype(jnp.float32)
        union_f = (S + col - row).astype(jnp.float32)
        iou_new = inter_f / union_f

        m_l0 = jnp.sum(row * col0_i)
        anyp0_i = jnp.where(((n_total - S) > 0) | (m_l0 > 0), 1, 0)
        prod_ok = col0_i * anyp0_i + (1 - col0_i) * jnp.where(row > 0, 1, 0)
        active_i = lp_i * prod_ok
        hits_i = active_i * jnp.where(iou_new > 0.5, 1, 0)

        n_active = jnp.sum(active_i)
        last = jnp.max(active_i * (ci + 1)) - 1
        iou_sel = jnp.sum(jnp.where(ci == last, iou_new, zero_f))
        iou = jnp.where(n_active > 0, iou_sel, iou)
        tp = tp + jnp.sum(hits_i)

        merged0 = jnp.sum(M * hits_i, axis=1, keepdims=True)
        M = M * (1 - hits_i) + col0_i * merged0
        ah_i = jnp.where(jnp.sum(hits_i) > 0, 1, 0)
        rowm = jnp.sum(M * is_row, axis=0, keepdims=True)
        M2 = M * (1 - is_row) + row0_i * rowm
        M = M + ah_i * (M2 - M)
        return (M, tp, iou)

    M, tp, iou = lax.fori_loop(
        0, NL, outer, (M0, jnp.int32(0), jnp.float32(0.0)))

    fn = jnp.sum(jnp.where(jnp.sum(M, axis=1, keepdims=True) > 0, 1, 0))
    fp = jnp.sum(jnp.where(jnp.sum(M, axis=0, keepdims=True) > 0, 1, 0))
    res = iou / (tp.astype(jnp.float32)
                 + jnp.float32(0.5) * (fn + fp).astype(jnp.float32))
    out_ref[...] = jnp.full((1, 1), res, jnp.float32)


def _sim_tc(parts):
    return pl.pallas_call(
        _sim_body,
        out_shape=jax.ShapeDtypeStruct((1, 1), jnp.float32),
    )(parts)


def kernel(gt, pred):
    gt = jnp.asarray(gt, jnp.int32).reshape(-1)
    pred = jnp.asarray(pred, jnp.int32).reshape(-1)
    parts = _hist_sc(gt, pred)
    return _sim_tc(parts)[0, 0]


# speedup vs baseline: 130.8036x; 1.3003x over previous
import dataclasses
import functools

import jax
import jax.numpy as jnp
from jax import lax
from jax.experimental import pallas as pl
from jax.experimental.pallas import tpu as pltpu
from jax.experimental.pallas import tpu_sc as plsc

NL = 20
ROWS, COLS = 1024, 1024
NPIX = ROWS * COLS
NC, NS, L = 2, 16, 16
NW = NC * NS
ROWS_W = ROWS // NW
VPR = COLS // L
NBINS = NL * NL
HC = NL * L


def _hist_sc(gt, pred):
    mesh = plsc.VectorSubcoreMesh(core_axis_name="c", subcore_axis_name="s")
    cp = pltpu.CompilerParams()
    if "needs_layout_passes" in pltpu.CompilerParams.__dataclass_fields__:
        cp = dataclasses.replace(cp, needs_layout_passes=False)

    @functools.partial(
        pl.kernel,
        out_type=jax.ShapeDtypeStruct((NW, NBINS, 1), jnp.int32),
        mesh=mesh,
        compiler_params=cp,
        scratch_types=[
            pltpu.VMEM((ROWS_W, COLS), jnp.int32),
            pltpu.VMEM((ROWS_W, COLS), jnp.int32),
            pltpu.VMEM((NL, HC), jnp.int32),
            pltpu.VMEM((NBINS, 1), jnp.int32),
        ],
    )
    def hist_kernel(gt_hbm, pred_hbm, out_hbm, gt_v, pred_v, hist_v, part_v):
        wid = lax.axis_index("s") * NC + lax.axis_index("c")
        base = wid * ROWS_W

        zeros = jnp.zeros((L,), jnp.int32)

        @pl.loop(0, NL)
        def _(g):
            @pl.loop(0, NL)
            def _(c):
                hist_v[g, pl.ds(c * L, L)] = zeros

        pltpu.sync_copy(gt_hbm.at[pl.ds(base, ROWS_W)], gt_v)
        pltpu.sync_copy(pred_hbm.at[pl.ds(base, ROWS_W)], pred_v)

        lane = lax.iota(jnp.int32, L)
        ones = jnp.ones((L,), jnp.int32)

        @pl.loop(0, ROWS_W)
        def _(r):
            @plsc.parallel_loop(0, VPR, unroll=4)
            def _(c):
                g = gt_v[r, pl.ds(c * L, L)]
                p = pred_v[r, pl.ds(c * L, L)]
                col = p * L + lane
                plsc.addupdate_scatter(hist_v, [g, col], ones)

        last_lane = lane == (L - 1)
        zeros_idx = jnp.zeros((L,), jnp.int32)

        @pl.loop(0, NL)
        def _(g):
            @pl.loop(0, NL)
            def _(p):
                v = hist_v[g, pl.ds(p * L, L)]
                tot = plsc.cumsum(v)
                b = jnp.full((L,), g * NL + p, jnp.int32)
                plsc.store_scatter(part_v, [b, zeros_idx], tot,
                                   mask=last_lane)

        pltpu.sync_copy(part_v, out_hbm.at[wid])

    return hist_kernel(gt, pred)


def _sim_body(parts_ref, out_ref):
    x = parts_ref[...]
    V = jnp.sum(x, axis=0).astype(jnp.float32)
    bi0 = lax.broadcasted_iota(jnp.int32, (NBINS, NL), 0)
    bi1 = lax.broadcasted_iota(jnp.int32, (NBINS, NL), 1)
    B = jnp.where(bi0 % NL == bi1, jnp.float32(1.0), jnp.float32(0.0))
    ai0 = lax.broadcasted_iota(jnp.int32, (NL, NBINS), 0)
    ai1 = lax.broadcasted_iota(jnp.int32, (NL, NBINS), 1)
    A = jnp.where(ai1 // NL == ai0, jnp.float32(1.0), jnp.float32(0.0))
    M0 = jnp.dot(A, V * B,
                 preferred_element_type=jnp.float32).astype(jnp.int32)
    rows0 = jnp.sum(M0, axis=1, keepdims=True)

    ci = lax.broadcasted_iota(jnp.int32, (1, NL), 1)
    ri = lax.broadcasted_iota(jnp.int32, (NL, 1), 0)
    col0_i = jnp.where(ci == 0, 1, 0)
    row0_i = jnp.where(ri == 0, 1, 0)
    n_total = jnp.int32(NPIX)
    zero_f = jnp.float32(0.0)

    def outer(label, carry):
        M, tp, iou = carry
        is_row = jnp.where(ri == label, 1, 0)
        row = jnp.sum(M * is_row, axis=0, keepdims=True)
        S = jnp.sum(row)
        col = jnp.sum(M, axis=0, keepdims=True)
        lp_i = jnp.where(jnp.sum(rows0 * is_row) > 0, 1, 0)

        inter_f = row.astype(jnp.float32)
        union_f = (S + col - row).astype(jnp.float32)
        iou_new = inter_f / union_f

        m_l0 = jnp.sum(row * col0_i)
        anyp0_i = jnp.where(((n_total - S) > 0) | (m_l0 > 0), 1, 0)
        prod_ok = col0_i * anyp0_i + (1 - col0_i) * jnp.where(row > 0, 1, 0)
        active_i = lp_i * prod_ok
        hits_i = active_i * jnp.where(iou_new > 0.5, 1, 0)

        n_active = jnp.sum(active_i)
        last = jnp.max(active_i * (ci + 1)) - 1
        iou_sel = jnp.sum(jnp.where(ci == last, iou_new, zero_f))
        iou = jnp.where(n_active > 0, iou_sel, iou)
        tp = tp + jnp.sum(hits_i)

        merged0 = jnp.sum(M * hits_i, axis=1, keepdims=True)
        M = M * (1 - hits_i) + col0_i * merged0
        ah_i = jnp.where(jnp.sum(hits_i) > 0, 1, 0)
        rowm = jnp.sum(M * is_row, axis=0, keepdims=True)
        M2 = M * (1 - is_row) + row0_i * rowm
        M = M + ah_i * (M2 - M)
        return (M, tp, iou)

    M, tp, iou = lax.fori_loop(
        0, NL, outer, (M0, jnp.int32(0), jnp.float32(0.0)))

    fn = jnp.sum(jnp.where(jnp.sum(M, axis=1, keepdims=True) > 0, 1, 0))
    fp = jnp.sum(jnp.where(jnp.sum(M, axis=0, keepdims=True) > 0, 1, 0))
    res = iou / (tp.astype(jnp.float32)
                 + jnp.float32(0.5) * (fn + fp).astype(jnp.float32))
    out_ref[...] = jnp.full((1, 1), res, jnp.float32)


def _sim_tc(parts):
    return pl.pallas_call(
        _sim_body,
        out_shape=jax.ShapeDtypeStruct((1, 1), jnp.float32),
    )(parts)


def kernel(gt, pred):
    gt = jnp.asarray(gt, jnp.int32)
    pred = jnp.asarray(pred, jnp.int32)
    parts = _hist_sc(gt, pred)
    return _sim_tc(parts)[0, 0]


# speedup vs baseline: 150.7152x; 1.1522x over previous
import dataclasses
import functools

import jax
import jax.numpy as jnp
from jax import lax
from jax.experimental import pallas as pl
from jax.experimental.pallas import tpu as pltpu
from jax.experimental.pallas import tpu_sc as plsc

NL = 20
ROWS, COLS = 1024, 1024
NPIX = ROWS * COLS
NC, NS, L = 2, 16, 16
NW = NC * NS
ROWS_W = ROWS // NW
VPR = COLS // L
NBINS = NL * NL
HC = NL * L
CH_ROWS = 8
NCHUNK = ROWS_W // CH_ROWS


def _hist_sc(gt, pred):
    mesh = plsc.VectorSubcoreMesh(core_axis_name="c", subcore_axis_name="s")
    cp = pltpu.CompilerParams()
    if "needs_layout_passes" in pltpu.CompilerParams.__dataclass_fields__:
        cp = dataclasses.replace(cp, needs_layout_passes=False)

    @functools.partial(
        pl.kernel,
        out_type=jax.ShapeDtypeStruct((NW, NL, NL), jnp.int32),
        mesh=mesh,
        compiler_params=cp,
        scratch_types=[
            pltpu.VMEM((2, CH_ROWS, COLS), jnp.int32),
            pltpu.VMEM((2, CH_ROWS, COLS), jnp.int32),
            pltpu.VMEM((NL, HC), jnp.int32),
            pltpu.VMEM((NL, NL), jnp.int32),
            pltpu.SemaphoreType.DMA,
            pltpu.SemaphoreType.DMA,
            pltpu.SemaphoreType.DMA,
            pltpu.SemaphoreType.DMA,
        ],
    )
    def hist_kernel(gt_hbm, pred_hbm, out_hbm, gt_v, pred_v, hist_v, part_v,
                    sg0, sg1, sp0, sp1):
        wid = lax.axis_index("s") * NC + lax.axis_index("c")
        base = wid * ROWS_W
        sg = (sg0, sg1)
        sp = (sp0, sp1)

        zeros = jnp.zeros((L,), jnp.int32)

        @pl.loop(0, NL)
        def _(g):
            @pl.loop(0, NL)
            def _(c):
                hist_v[g, pl.ds(c * L, L)] = zeros

        lane = lax.iota(jnp.int32, L)
        ones = jnp.ones((L,), jnp.int32)

        def start(k):
            b = k % 2
            rows = pl.ds(base + k * CH_ROWS, CH_ROWS)
            cg = pltpu.async_copy(gt_hbm.at[rows], gt_v.at[b], sg[b])
            cpv = pltpu.async_copy(pred_hbm.at[rows], pred_v.at[b], sp[b])
            return cg, cpv

        pending = start(0)
        for k in range(NCHUNK):
            b = k % 2
            for h in pending:
                h.wait()
            if k + 1 < NCHUNK:
                pending = start(k + 1)

            @pl.loop(0, CH_ROWS)
            def _(r):
                @plsc.parallel_loop(0, VPR, unroll=4)
                def _(c):
                    g = gt_v[b, r, pl.ds(c * L, L)]
                    p = pred_v[b, r, pl.ds(c * L, L)]
                    col = p * L + lane
                    plsc.addupdate_scatter(hist_v, [g, col], ones)

        last_lane = lane == (L - 1)

        @pl.loop(0, NL)
        def _(g):
            @pl.loop(0, NL)
            def _(p):
                v = hist_v[g, pl.ds(p * L, L)]
                tot = plsc.cumsum(v)
                gidx = jnp.full((L,), g, jnp.int32)
                pidx = jnp.full((L,), p, jnp.int32)
                plsc.store_scatter(part_v, [gidx, pidx], tot,
                                   mask=last_lane)

        pltpu.sync_copy(part_v, out_hbm.at[wid])

    return hist_kernel(gt, pred)


def _sim_body(parts_ref, out_ref):
    x = parts_ref[...]
    M0 = jnp.sum(x, axis=0)
    rows0 = jnp.sum(M0, axis=1, keepdims=True)

    ci = lax.broadcasted_iota(jnp.int32, (1, NL), 1)
    ri = lax.broadcasted_iota(jnp.int32, (NL, 1), 0)
    col0_i = jnp.where(ci == 0, 1, 0)
    row0_i = jnp.where(ri == 0, 1, 0)
    n_total = jnp.int32(NPIX)
    zero_f = jnp.float32(0.0)

    def outer(label, carry):
        M, tp, iou = carry
        is_row = jnp.where(ri == label, 1, 0)
        row = jnp.sum(M * is_row, axis=0, keepdims=True)
        S = jnp.sum(row)
        col = jnp.sum(M, axis=0, keepdims=True)
        lp_i = jnp.where(jnp.sum(rows0 * is_row) > 0, 1, 0)

        inter_f = row.astype(jnp.float32)
        union_f = (S + col - row).astype(jnp.float32)
        iou_new = inter_f / union_f

        m_l0 = jnp.sum(row * col0_i)
        anyp0_i = jnp.where(((n_total - S) > 0) | (m_l0 > 0), 1, 0)
        prod_ok = col0_i * anyp0_i + (1 - col0_i) * jnp.where(row > 0, 1, 0)
        active_i = lp_i * prod_ok
        hits_i = active_i * jnp.where(iou_new > 0.5, 1, 0)

        n_active = jnp.sum(active_i)
        last = jnp.max(active_i * (ci + 1)) - 1
        iou_sel = jnp.sum(jnp.where(ci == last, iou_new, zero_f))
        iou = jnp.where(n_active > 0, iou_sel, iou)
        tp = tp + jnp.sum(hits_i)

        merged0 = jnp.sum(M * hits_i, axis=1, keepdims=True)
        M = M * (1 - hits_i) + col0_i * merged0
        ah_i = jnp.where(jnp.sum(hits_i) > 0, 1, 0)
        rowm = jnp.sum(M * is_row, axis=0, keepdims=True)
        M2 = M * (1 - is_row) + row0_i * rowm
        M = M + ah_i * (M2 - M)
        return (M, tp, iou)

    M, tp, iou = lax.fori_loop(
        0, NL, outer, (M0, jnp.int32(0), jnp.float32(0.0)))

    fn = jnp.sum(jnp.where(jnp.sum(M, axis=1, keepdims=True) > 0, 1, 0))
    fp = jnp.sum(jnp.where(jnp.sum(M, axis=0, keepdims=True) > 0, 1, 0))
    res = iou / (tp.astype(jnp.float32)
                 + jnp.float32(0.5) * (fn + fp).astype(jnp.float32))
    out_ref[...] = jnp.full((1, 1), res, jnp.float32)


def _sim_tc(parts):
    return pl.pallas_call(
        _sim_body,
        out_shape=jax.ShapeDtypeStruct((1, 1), jnp.float32),
    )(parts)


def kernel(gt, pred):
    gt = jnp.asarray(gt, jnp.int32)
    pred = jnp.asarray(pred, jnp.int32)
    parts = _hist_sc(gt, pred)
    return _sim_tc(parts)[0, 0]


# speedup vs baseline: 189.5488x; 1.2577x over previous
import dataclasses
import functools

import jax
import jax.numpy as jnp
from jax import lax
from jax.experimental import pallas as pl
from jax.experimental.pallas import tpu as pltpu
from jax.experimental.pallas import tpu_sc as plsc

NL = 20
ROWS, COLS = 1024, 1024
NPIX = ROWS * COLS
NC, NS, L = 2, 16, 16
NW = NC * NS
ROWS_W = ROWS // NW
VPR = COLS // L
NBINS = NL * NL
HC = NL * L
CH_ROWS = 8
NCHUNK = ROWS_W // CH_ROWS


def _hist_sc(gt, pred):
    mesh = plsc.VectorSubcoreMesh(core_axis_name="c", subcore_axis_name="s")
    cp = pltpu.CompilerParams()
    if "needs_layout_passes" in pltpu.CompilerParams.__dataclass_fields__:
        cp = dataclasses.replace(cp, needs_layout_passes=False)

    @functools.partial(
        pl.kernel,
        out_type=jax.ShapeDtypeStruct((NW, NL, NL), jnp.int32),
        mesh=mesh,
        compiler_params=cp,
        scratch_types=[
            pltpu.VMEM((2, CH_ROWS, COLS), jnp.int32),
            pltpu.VMEM((2, CH_ROWS, COLS), jnp.int32),
            pltpu.VMEM((NL, HC), jnp.int32),
            pltpu.VMEM((NL, NL), jnp.int32),
            pltpu.SemaphoreType.DMA,
            pltpu.SemaphoreType.DMA,
            pltpu.SemaphoreType.DMA,
            pltpu.SemaphoreType.DMA,
        ],
    )
    def hist_kernel(gt_hbm, pred_hbm, out_hbm, gt_v, pred_v, hist_v, part_v,
                    sg0, sg1, sp0, sp1):
        wid = lax.axis_index("s") * NC + lax.axis_index("c")
        base = wid * ROWS_W
        sg = (sg0, sg1)
        sp = (sp0, sp1)

        zeros = jnp.zeros((L,), jnp.int32)

        @pl.loop(0, NL)
        def _(g):
            @pl.loop(0, NL)
            def _(c):
                hist_v[g, pl.ds(c * L, L)] = zeros

        lane = lax.iota(jnp.int32, L)
        ones = jnp.ones((L,), jnp.int32)

        def start(k):
            b = k % 2
            rows = pl.ds(base + k * CH_ROWS, CH_ROWS)
            cg = pltpu.async_copy(gt_hbm.at[rows], gt_v.at[b], sg[b])
            cpv = pltpu.async_copy(pred_hbm.at[rows], pred_v.at[b], sp[b])
            return cg, cpv

        pending = start(0)
        for k in range(NCHUNK):
            b = k % 2
            for h in pending:
                h.wait()
            if k + 1 < NCHUNK:
                pending = start(k + 1)

            @pl.loop(0, CH_ROWS)
            def _(r):
                @plsc.parallel_loop(0, VPR, unroll=4)
                def _(c):
                    g = gt_v[b, r, pl.ds(c * L, L)]
                    p = pred_v[b, r, pl.ds(c * L, L)]
                    col = p * L + lane
                    plsc.addupdate_scatter(hist_v, [g, col], ones)

        last_lane = lane == (L - 1)

        @pl.loop(0, NL)
        def _(g):
            @pl.loop(0, NL)
            def _(p):
                v = hist_v[g, pl.ds(p * L, L)]
                tot = plsc.cumsum(v)
                gidx = jnp.full((L,), g, jnp.int32)
                pidx = jnp.full((L,), p, jnp.int32)
                plsc.store_scatter(part_v, [gidx, pidx], tot,
                                   mask=last_lane)

        pltpu.sync_copy(part_v, out_hbm.at[wid])

    return hist_kernel(gt, pred)


def _sim_body(parts_ref, out_ref):
    x = parts_ref[...]
    M0 = jnp.sum(x, axis=0)
    S0 = jnp.sum(M0, axis=1, keepdims=True)
    C0 = jnp.sum(M0, axis=0, keepdims=True)
    lp = jnp.where(S0 > 0, 1, 0)

    ci = lax.broadcasted_iota(jnp.int32, (1, NL), 1)
    ri = lax.broadcasted_iota(jnp.int32, (NL, 1), 0)
    col0 = jnp.where(ci == 0, 1, 0)
    row0 = jnp.where(ri == 0, 1, 0)
    pmask = 1 - col0
    vmask = 1 - row0
    n_total = jnp.int32(NPIX)

    U = S0 + C0 - M0
    Hm1 = lp * jnp.where(2 * M0 > U, 1, 0) * pmask
    hitp = jnp.sum(Hm1, axis=0, keepdims=True)
    lhit = jnp.sum(Hm1 * ri, axis=0, keepdims=True)
    before = hitp * jnp.where(lhit < ri, 1, 0)

    m_col0 = jnp.sum(M0 * col0, axis=1, keepdims=True)
    inter0 = m_col0 + jnp.sum(M0 * before, axis=1, keepdims=True)
    c00 = jnp.sum(C0 * col0)
    c0cur = c00 + jnp.sum(C0 * before, axis=1, keepdims=True)
    U0 = S0 + c0cur - inter0
    anyp0 = jnp.where(((n_total - S0) > 0) | (inter0 > 0), 1, 0)
    active0 = lp * anyp0
    hit0 = active0 * jnp.where(2 * inter0 > U0, 1, 0)
    tp = jnp.sum(Hm1) + jnp.sum(hit0)

    actp = lp * jnp.where(M0 > 0, 1, 0) * pmask * (1 - before)
    ACT = actp + active0 * col0
    key = ri * NL + ci
    km = jnp.max(ACT * (key + 1)) - 1
    sel = jnp.where(key == km, 1, 0)
    INTM = M0 * pmask + inter0 * col0
    UM = U * pmask + U0 * col0
    inter_sel = jnp.sum(INTM * sel).astype(jnp.float32)
    union_sel = jnp.sum(UM * sel).astype(jnp.float32)
    iou = jnp.where(km >= 0, inter_sel / union_sel, jnp.float32(0.0))

    HITS = Hm1 + hit0 * col0
    rowhit = jnp.where(jnp.sum(HITS, axis=1, keepdims=True) > 0, 1, 0)
    fn = (jnp.sum(jnp.where(S0 > 0, 1, 0) * (1 - rowhit) * vmask)
          + jnp.where(jnp.sum(S0 * row0)
                      + jnp.sum(S0 * rowhit * vmask) > 0, 1, 0))
    fp = (jnp.sum(jnp.where(C0 > 0, 1, 0) * (1 - hitp) * pmask)
          + jnp.where(c00 + jnp.sum(C0 * hitp * pmask) > 0, 1, 0))

    res = iou / (tp.astype(jnp.float32)
                 + jnp.float32(0.5) * (fn + fp).astype(jnp.float32))
    out_ref[...] = jnp.full((1, 1), res, jnp.float32)


def _sim_tc(parts):
    return pl.pallas_call(
        _sim_body,
        out_shape=jax.ShapeDtypeStruct((1, 1), jnp.float32),
    )(parts)


def kernel(gt, pred):
    gt = jnp.asarray(gt, jnp.int32)
    pred = jnp.asarray(pred, jnp.int32)
    parts = _hist_sc(gt, pred)
    return _sim_tc(parts)[0, 0]


# speedup vs baseline: 215.7642x; 1.1383x over previous
import dataclasses
import functools

import jax
import jax.numpy as jnp
from jax import lax
from jax.experimental import pallas as pl
from jax.experimental.pallas import tpu as pltpu
from jax.experimental.pallas import tpu_sc as plsc

NL = 20
ROWS, COLS = 1024, 1024
NPIX = ROWS * COLS
NC, NS, L = 2, 16, 16
NW = NC * NS
ROWS_W = ROWS // NW
VPR = COLS // L
NBINS = NL * NL
HC = NL * L
CH_ROWS = 8
NCHUNK = ROWS_W // CH_ROWS


def _hist_sc(gt, pred):
    mesh = plsc.VectorSubcoreMesh(core_axis_name="c", subcore_axis_name="s")
    cp = pltpu.CompilerParams()
    if "needs_layout_passes" in pltpu.CompilerParams.__dataclass_fields__:
        cp = dataclasses.replace(cp, needs_layout_passes=False)

    @functools.partial(
        pl.kernel,
        out_type=jax.ShapeDtypeStruct((NW, NL, NL), jnp.int32),
        mesh=mesh,
        compiler_params=cp,
        scratch_types=[
            pltpu.VMEM((2, CH_ROWS, COLS), jnp.int32),
            pltpu.VMEM((2, CH_ROWS, COLS), jnp.int32),
            pltpu.VMEM((NL, HC), jnp.int32),
            pltpu.VMEM((NL, NL), jnp.int32),
            pltpu.SemaphoreType.DMA,
            pltpu.SemaphoreType.DMA,
            pltpu.SemaphoreType.DMA,
            pltpu.SemaphoreType.DMA,
        ],
    )
    def hist_kernel(gt_hbm, pred_hbm, out_hbm, gt_v, pred_v, hist_v, part_v,
                    sg0, sg1, sp0, sp1):
        wid = lax.axis_index("s") * NC + lax.axis_index("c")
        base = wid * ROWS_W
        sg = (sg0, sg1)
        sp = (sp0, sp1)

        zeros = jnp.zeros((L,), jnp.int32)
        lane = lax.iota(jnp.int32, L)
        ones = jnp.ones((L,), jnp.int32)

        def start(k):
            b = k % 2
            rows = pl.ds(base + k * CH_ROWS, CH_ROWS)
            cg = pltpu.async_copy(gt_hbm.at[rows], gt_v.at[b], sg[b])
            cpv = pltpu.async_copy(pred_hbm.at[rows], pred_v.at[b], sp[b])
            return cg, cpv

        pending = start(0)

        @pl.loop(0, NL)
        def _(g):
            @plsc.parallel_loop(0, NL, unroll=4)
            def _(c):
                hist_v[g, pl.ds(c * L, L)] = zeros

        for k in range(NCHUNK):
            b = k % 2
            for h in pending:
                h.wait()
            if k + 1 < NCHUNK:
                pending = start(k + 1)

            @pl.loop(0, CH_ROWS)
            def _(r):
                @plsc.parallel_loop(0, VPR, unroll=8)
                def _(c):
                    g = gt_v[b, r, pl.ds(c * L, L)]
                    p = pred_v[b, r, pl.ds(c * L, L)]
                    col = p * L + lane
                    plsc.addupdate_scatter(hist_v, [g, col], ones)

        last_lane = lane == (L - 1)

        @pl.loop(0, NL)
        def _(g):
            @plsc.parallel_loop(0, NL, unroll=4)
            def _(p):
                v = hist_v[g, pl.ds(p * L, L)]
                tot = plsc.cumsum(v)
                gidx = jnp.full((L,), g, jnp.int32)
                pidx = jnp.full((L,), p, jnp.int32)
                plsc.store_scatter(part_v, [gidx, pidx], tot,
                                   mask=last_lane)

        pltpu.sync_copy(part_v, out_hbm.at[wid])

    return hist_kernel(gt, pred)


def _sim_body(parts_ref, out_ref):
    x = parts_ref[...]
    M0 = jnp.sum(x, axis=0)
    S0 = jnp.sum(M0, axis=1, keepdims=True)
    C0 = jnp.sum(M0, axis=0, keepdims=True)
    lp = jnp.where(S0 > 0, 1, 0)

    ci = lax.broadcasted_iota(jnp.int32, (1, NL), 1)
    ri = lax.broadcasted_iota(jnp.int32, (NL, 1), 0)
    col0 = jnp.where(ci == 0, 1, 0)
    row0 = jnp.where(ri == 0, 1, 0)
    pmask = 1 - col0
    vmask = 1 - row0
    n_total = jnp.int32(NPIX)

    U = S0 + C0 - M0
    Hm1 = lp * jnp.where(2 * M0 > U, 1, 0) * pmask
    hitp = jnp.sum(Hm1, axis=0, keepdims=True)
    lhit = jnp.sum(Hm1 * ri, axis=0, keepdims=True)
    before = hitp * jnp.where(lhit < ri, 1, 0)

    m_col0 = jnp.sum(M0 * col0, axis=1, keepdims=True)
    inter0 = m_col0 + jnp.sum(M0 * before, axis=1, keepdims=True)
    c00 = jnp.sum(C0 * col0)
    c0cur = c00 + jnp.sum(C0 * before, axis=1, keepdims=True)
    U0 = S0 + c0cur - inter0
    anyp0 = jnp.where(((n_total - S0) > 0) | (inter0 > 0), 1, 0)
    active0 = lp * anyp0
    hit0 = active0 * jnp.where(2 * inter0 > U0, 1, 0)
    tp = jnp.sum(Hm1) + jnp.sum(hit0)

    actp = lp * jnp.where(M0 > 0, 1, 0) * pmask * (1 - before)
    ACT = actp + active0 * col0
    key = ri * NL + ci
    km = jnp.max(ACT * (key + 1)) - 1
    sel = jnp.where(key == km, 1, 0)
    INTM = M0 * pmask + inter0 * col0
    UM = U * pmask + U0 * col0
    inter_sel = jnp.sum(INTM * sel).astype(jnp.float32)
    union_sel = jnp.sum(UM * sel).astype(jnp.float32)
    iou = jnp.where(km >= 0, inter_sel / union_sel, jnp.float32(0.0))

    HITS = Hm1 + hit0 * col0
    rowhit = jnp.where(jnp.sum(HITS, axis=1, keepdims=True) > 0, 1, 0)
    fn = (jnp.sum(jnp.where(S0 > 0, 1, 0) * (1 - rowhit) * vmask)
          + jnp.where(jnp.sum(S0 * row0)
                      + jnp.sum(S0 * rowhit * vmask) > 0, 1, 0))
    fp = (jnp.sum(jnp.where(C0 > 0, 1, 0) * (1 - hitp) * pmask)
          + jnp.where(c00 + jnp.sum(C0 * hitp * pmask) > 0, 1, 0))

    res = iou / (tp.astype(jnp.float32)
                 + jnp.float32(0.5) * (fn + fp).astype(jnp.float32))
    out_ref[...] = jnp.full((1, 1), res, jnp.float32)


def _sim_tc(parts):
    return pl.pallas_call(
        _sim_body,
        out_shape=jax.ShapeDtypeStruct((1, 1), jnp.float32),
    )(parts)


def kernel(gt, pred):
    gt = jnp.asarray(gt, jnp.int32)
    pred = jnp.asarray(pred, jnp.int32)
    parts = _hist_sc(gt, pred)
    return _sim_tc(parts)[0, 0]
